# Initial kernel scaffold; baseline (speedup 1.0000x reference)
import jax
import jax.numpy as jnp
from jax.experimental import pallas as pl


def kernel(features, edge_index, edge_types, W_self, W_neigh, b_conv, W_pred, b_pred):
    raise NotImplementedError("write your pallas kernel here")


# speedup vs baseline: 9.8521x; 9.8521x over previous
import functools

import jax
import jax.numpy as jnp
from jax import lax
from jax.experimental import pallas as pl
from jax.experimental.pallas import tpu as pltpu
from jax.experimental.pallas import tpu_sc as plsc

N_NODES = 10000
N_EDGES = 320000
D = 128

NC = 2
NS = 16
NW = NC * NS
K = 128
NCHUNK = 10112 // K
EPT = NCHUNK * K
E_PAD = NW * EPT
NPAD = 10112
ROWS_PER_SUB = NPAD // NS


def _sc_mesh():
    return plsc.VectorSubcoreMesh(core_axis_name="c", subcore_axis_name="s")


@functools.partial(
    pl.kernel,
    out_type=(
        jax.ShapeDtypeStruct((NC, NPAD, D), jnp.float32),
        jax.ShapeDtypeStruct((NW, NPAD), jnp.float32),
    ),
    mesh=_sc_mesh(),
    scratch_types=[
        pltpu.VMEM_SHARED((NPAD, D), jnp.float32),
        pltpu.VMEM((NCHUNK, K), jnp.int32),
        pltpu.VMEM((NCHUNK, K), jnp.int32),
        pltpu.VMEM((K, D), jnp.float32),
        pltpu.VMEM((NPAD,), jnp.float32),
        pltpu.SemaphoreType.DMA,
    ],
    compiler_params=pltpu.CompilerParams(needs_layout_passes=False),
)
def _aggregate(feat_hbm, srcw_hbm, dstw_hbm, zrow_hbm, z1_hbm,
               agg_hbm, deg_hbm,
               agg_sh, src_v, dst_v, rows_v, deg_v, sem):
    c = lax.axis_index("c")
    s = lax.axis_index("s")
    wid = s * NC + c

    pltpu.sync_copy(zrow_hbm.at[pl.ds(s * ROWS_PER_SUB, ROWS_PER_SUB)],
                    agg_sh.at[pl.ds(s * ROWS_PER_SUB, ROWS_PER_SUB)])
    pltpu.sync_copy(z1_hbm, deg_v)
    pltpu.sync_copy(srcw_hbm.at[wid], src_v)
    pltpu.sync_copy(dstw_hbm.at[wid], dst_v)
    plsc.subcore_barrier()

    ones = jnp.ones((16,), jnp.float32)

    def chunk(j, _):
        pltpu.async_copy(feat_hbm.at[src_v.at[j]], rows_v, sem).wait()
        pltpu.sync_copy(rows_v, agg_sh.at[dst_v.at[j]], add=True)
        for i in range(K // 16):
            idx = dst_v[j, pl.ds(i * 16, 16)]
            plsc.addupdate_scatter(deg_v, [idx], ones)
        return 0

    lax.fori_loop(0, NCHUNK, chunk, 0)

    pltpu.sync_copy(deg_v, deg_hbm.at[wid])
    plsc.subcore_barrier()
    pltpu.sync_copy(agg_sh.at[pl.ds(s * ROWS_PER_SUB, ROWS_PER_SUB)],
                    agg_hbm.at[c, pl.ds(s * ROWS_PER_SUB, ROWS_PER_SUB)])


def _dense_body(feat, agg, degp, ws, wn, bc, wp, bp, s_out, t_out):
    aggs = agg[0] + agg[1]
    deg = jnp.sum(degp[...], axis=0)
    hn = aggs * (1.0 / jnp.maximum(deg, 1.0))[:, None]
    h = (jnp.dot(feat[...], ws[...], preferred_element_type=jnp.float32)
         + jnp.dot(hn, wn[...], preferred_element_type=jnp.float32)
         + bc[...])
    x = jnp.maximum(h, 0.0)
    b = bp[0]
    s_out[...] = jnp.sum(x * wp[0:1, :], axis=1) + b
    t_out[...] = jnp.sum(x * wp[1:2, :], axis=1) + b


def _dense(feat_pad, agg, degp, ws, wn, bc2, wp2, bp):
    return pl.pallas_call(
        _dense_body,
        out_shape=(
            jax.ShapeDtypeStruct((NPAD,), jnp.float32),
            jax.ShapeDtypeStruct((NPAD,), jnp.float32),
        ),
        in_specs=[
            pl.BlockSpec(memory_space=pltpu.VMEM),
            pl.BlockSpec(memory_space=pltpu.VMEM),
            pl.BlockSpec(memory_space=pltpu.VMEM),
            pl.BlockSpec(memory_space=pltpu.VMEM),
            pl.BlockSpec(memory_space=pltpu.VMEM),
            pl.BlockSpec(memory_space=pltpu.VMEM),
            pl.BlockSpec(memory_space=pltpu.VMEM),
            pl.BlockSpec(memory_space=pltpu.SMEM),
        ],
        out_specs=(
            pl.BlockSpec(memory_space=pltpu.VMEM),
            pl.BlockSpec(memory_space=pltpu.VMEM),
        ),
        compiler_params=pltpu.CompilerParams(
            vmem_limit_bytes=100 * 1024 * 1024,
        ),
    )(feat_pad, agg, degp, ws, wn, bc2, wp2, bp)


@functools.partial(
    pl.kernel,
    out_type=jax.ShapeDtypeStruct((NW, NCHUNK, K), jnp.float32),
    mesh=_sc_mesh(),
    scratch_types=[
        pltpu.VMEM((NPAD,), jnp.float32),
        pltpu.VMEM((NPAD,), jnp.float32),
        pltpu.VMEM((NCHUNK, K), jnp.int32),
        pltpu.VMEM((NCHUNK, K), jnp.int32),
        pltpu.VMEM((NCHUNK, K), jnp.float32),
    ],
    compiler_params=pltpu.CompilerParams(needs_layout_passes=False),
)
def _edge_logits(s_hbm, t_hbm, srcw_hbm, dstw_hbm, out_hbm,
                 s_v, t_v, src_v, dst_v, out_v):
    c = lax.axis_index("c")
    s = lax.axis_index("s")
    wid = s * NC + c

    pltpu.sync_copy(s_hbm, s_v)
    pltpu.sync_copy(t_hbm, t_v)
    pltpu.sync_copy(srcw_hbm.at[wid], src_v)
    pltpu.sync_copy(dstw_hbm.at[wid], dst_v)

    def chunk(j, _):
        for i in range(K // 16):
            si = src_v[j, pl.ds(i * 16, 16)]
            di = dst_v[j, pl.ds(i * 16, 16)]
            z = plsc.load_gather(s_v, [si]) + plsc.load_gather(t_v, [di])
            out_v[j, pl.ds(i * 16, 16)] = 1.0 / (1.0 + jnp.exp(-z))
        return 0

    lax.fori_loop(0, NCHUNK, chunk, 0)
    pltpu.sync_copy(out_v, out_hbm.at[wid])


def kernel(features, edge_index, edge_types, W_self, W_neigh, b_conv,
           W_pred, b_pred):
    del edge_types
    src = edge_index[0].astype(jnp.int32)
    dst = edge_index[1].astype(jnp.int32)

    pad = E_PAD - N_EDGES
    fill = jnp.full((pad,), N_NODES, jnp.int32)
    srcw = jnp.concatenate([src, fill]).reshape(NW, NCHUNK, K)
    dstw = jnp.concatenate([dst, fill]).reshape(NW, NCHUNK, K)

    feat_pad = jnp.zeros((NPAD, D), jnp.float32).at[:N_NODES].set(features)
    zrow = jnp.zeros((NPAD, D), jnp.float32)
    z1 = jnp.zeros((NPAD,), jnp.float32)

    agg, degp = _aggregate(feat_pad, srcw, dstw, zrow, z1)

    wp2 = W_pred.reshape(2, D)
    bc2 = b_conv.reshape(1, D)
    s_arr, t_arr = _dense(feat_pad, agg, degp, W_self, W_neigh, bc2, wp2,
                          b_pred)

    logits = _edge_logits(s_arr, t_arr, srcw, dstw)
    return logits.reshape(-1)[:N_EDGES]


# speedup vs baseline: 10.2576x; 1.0412x over previous
import functools

import jax
import jax.numpy as jnp
from jax import lax
from jax.experimental import pallas as pl
from jax.experimental.pallas import tpu as pltpu
from jax.experimental.pallas import tpu_sc as plsc

N_NODES = 10000
N_EDGES = 320000
D = 128

NC = 2
NS = 16
NW = NC * NS
K = 128
NCHUNK = 79
EPT = NCHUNK * K
E_PAD = NW * EPT
NPAD = 10112
ROWS_PER_SUB = NPAD // NS


def _sc_mesh():
    return plsc.VectorSubcoreMesh(core_axis_name="c", subcore_axis_name="s")


@functools.partial(
    pl.kernel,
    out_type=(
        jax.ShapeDtypeStruct((NC, NPAD, D), jnp.float32),
        jax.ShapeDtypeStruct((NC * NPAD,), jnp.float32),
    ),
    mesh=_sc_mesh(),
    scratch_types=[
        pltpu.VMEM_SHARED((NPAD, D), jnp.float32),
        pltpu.VMEM_SHARED((NPAD,), jnp.float32),
        pltpu.VMEM((K,), jnp.int32),
        pltpu.VMEM((K,), jnp.int32),
        pltpu.VMEM((K,), jnp.int32),
        pltpu.VMEM((K,), jnp.int32),
        pltpu.VMEM((K, D), jnp.float32),
        pltpu.VMEM((K, D), jnp.float32),
        pltpu.VMEM((K,), jnp.float32),
        pltpu.VMEM((ROWS_PER_SUB,), jnp.float32),
        pltpu.SemaphoreType.DMA,
        pltpu.SemaphoreType.DMA,
        pltpu.SemaphoreType.DMA,
        pltpu.SemaphoreType.DMA,
    ],
    compiler_params=pltpu.CompilerParams(needs_layout_passes=False),
)
def _aggregate(feat_hbm, srcf_hbm, dstf_hbm, zrow_hbm, z1_hbm,
               agg_hbm, deg_hbm,
               agg_sh, deg_sh, src0, dst0, src1, dst1, rows_a, rows_b,
               ones_v, zbuf_v, ra, rb, si0, si1):
    c = lax.axis_index("c")
    s = lax.axis_index("s")
    wid = s * NC + c
    row0 = pl.multiple_of(s * ROWS_PER_SUB, 8)
    ebase = wid * EPT

    pltpu.sync_copy(zrow_hbm.at[pl.ds(row0, ROWS_PER_SUB)],
                    agg_sh.at[pl.ds(row0, ROWS_PER_SUB)])
    pltpu.sync_copy(z1_hbm.at[pl.ds(row0, ROWS_PER_SUB)], zbuf_v)
    pltpu.sync_copy(zbuf_v, deg_sh.at[pl.ds(row0, ROWS_PER_SUB)])
    for i in range(K // 16):
        ones_v[pl.ds(i * 16, 16)] = jnp.ones((16,), jnp.float32)
    plsc.subcore_barrier()

    def fetch(j, sb, db, sem):
        off = pl.multiple_of(ebase + j * K, 8)
        pltpu.async_copy(srcf_hbm.at[pl.ds(off, K)], sb, sem)
        pltpu.async_copy(dstf_hbm.at[pl.ds(off, K)], db, sem)

    def fetch_wait(j, sb, db, sem):
        off = pl.multiple_of(ebase + j * K, 8)
        pltpu.make_async_copy(srcf_hbm.at[pl.ds(off, K)], sb, sem).wait()
        pltpu.make_async_copy(dstf_hbm.at[pl.ds(off, K)], db, sem).wait()

    def gather(sb, rows, sem):
        pltpu.async_copy(feat_hbm.at[sb], rows, sem)

    def drain(db, rows, sem):
        pltpu.make_async_copy(feat_hbm.at[src0], rows, sem).wait()
        pltpu.sync_copy(rows, agg_sh.at[db], add=True)
        pltpu.sync_copy(ones_v, deg_sh.at[db], add=True)

    fetch(0, src0, dst0, si0)
    fetch_wait(0, src0, dst0, si0)
    gather(src0, rows_a, ra)
    fetch(1, src1, dst1, si1)

    def two_chunks(g, _):
        j = 2 * g
        fetch_wait(j + 1, src1, dst1, si1)
        gather(src1, rows_b, rb)
        drain(dst0, rows_a, ra)
        fetch(j + 2, src0, dst0, si0)
        fetch_wait(j + 2, src0, dst0, si0)
        gather(src0, rows_a, ra)
        drain(dst1, rows_b, rb)
        fetch(j + 3, src1, dst1, si1)
        return 0

    lax.fori_loop(0, (NCHUNK - 3) // 2, two_chunks, 0)
    fetch_wait(NCHUNK - 2, src1, dst1, si1)
    gather(src1, rows_b, rb)
    drain(dst0, rows_a, ra)
    fetch(NCHUNK - 1, src0, dst0, si0)
    fetch_wait(NCHUNK - 1, src0, dst0, si0)
    gather(src0, rows_a, ra)
    drain(dst1, rows_b, rb)
    drain(dst0, rows_a, ra)

    plsc.subcore_barrier()
    pltpu.sync_copy(agg_sh.at[pl.ds(row0, ROWS_PER_SUB)],
                    agg_hbm.at[c, pl.ds(row0, ROWS_PER_SUB)])
    doff = pl.multiple_of(c * NPAD + row0, 8)
    pltpu.sync_copy(deg_sh.at[pl.ds(row0, ROWS_PER_SUB)], zbuf_v)
    pltpu.sync_copy(zbuf_v, deg_hbm.at[pl.ds(doff, ROWS_PER_SUB)])


def _dense_body(feat, agg, degp, ws, wn, bc, wp, bp, s_out, t_out):
    aggs = agg[0] + agg[1]
    deg = degp[0] + degp[1]
    hn = aggs * (1.0 / jnp.maximum(deg, 1.0))[:, None]
    h = (jnp.dot(feat[...], ws[...], preferred_element_type=jnp.float32)
         + jnp.dot(hn, wn[...], preferred_element_type=jnp.float32)
         + bc[...])
    x = jnp.maximum(h, 0.0)
    b = bp[0]
    s_out[...] = jnp.sum(x * wp[0:1, :], axis=1) + b
    t_out[...] = jnp.sum(x * wp[1:2, :], axis=1) + b


def _dense(feat_pad, agg, degp, ws, wn, bc2, wp2, bp):
    return pl.pallas_call(
        _dense_body,
        out_shape=(
            jax.ShapeDtypeStruct((NPAD,), jnp.float32),
            jax.ShapeDtypeStruct((NPAD,), jnp.float32),
        ),
        in_specs=[
            pl.BlockSpec(memory_space=pltpu.VMEM),
            pl.BlockSpec(memory_space=pltpu.VMEM),
            pl.BlockSpec(memory_space=pltpu.VMEM),
            pl.BlockSpec(memory_space=pltpu.VMEM),
            pl.BlockSpec(memory_space=pltpu.VMEM),
            pl.BlockSpec(memory_space=pltpu.VMEM),
            pl.BlockSpec(memory_space=pltpu.VMEM),
            pl.BlockSpec(memory_space=pltpu.SMEM),
        ],
        out_specs=(
            pl.BlockSpec(memory_space=pltpu.VMEM),
            pl.BlockSpec(memory_space=pltpu.VMEM),
        ),
        compiler_params=pltpu.CompilerParams(
            vmem_limit_bytes=100 * 1024 * 1024,
        ),
    )(feat_pad, agg, degp, ws, wn, bc2, wp2, bp)


@functools.partial(
    pl.kernel,
    out_type=jax.ShapeDtypeStruct((NW, NCHUNK, K), jnp.float32),
    mesh=_sc_mesh(),
    scratch_types=[
        pltpu.VMEM((NPAD,), jnp.float32),
        pltpu.VMEM((NPAD,), jnp.float32),
        pltpu.VMEM((NCHUNK, K), jnp.int32),
        pltpu.VMEM((NCHUNK, K), jnp.int32),
        pltpu.VMEM((NCHUNK, K), jnp.float32),
    ],
    compiler_params=pltpu.CompilerParams(needs_layout_passes=False),
)
def _edge_logits(s_hbm, t_hbm, srcw_hbm, dstw_hbm, out_hbm,
                 s_v, t_v, src_v, dst_v, out_v):
    c = lax.axis_index("c")
    s = lax.axis_index("s")
    wid = s * NC + c

    pltpu.sync_copy(s_hbm, s_v)
    pltpu.sync_copy(t_hbm, t_v)
    pltpu.sync_copy(srcw_hbm.at[wid], src_v)
    pltpu.sync_copy(dstw_hbm.at[wid], dst_v)

    def chunk(j, _):
        for i in range(K // 16):
            si = src_v[j, pl.ds(i * 16, 16)]
            di = dst_v[j, pl.ds(i * 16, 16)]
            z = plsc.load_gather(s_v, [si]) + plsc.load_gather(t_v, [di])
            out_v[j, pl.ds(i * 16, 16)] = 1.0 / (1.0 + jnp.exp(-z))
        return 0

    lax.fori_loop(0, NCHUNK, chunk, 0)
    pltpu.sync_copy(out_v, out_hbm.at[wid])


def kernel(features, edge_index, edge_types, W_self, W_neigh, b_conv,
           W_pred, b_pred):
    del edge_types
    src = edge_index[0].astype(jnp.int32)
    dst = edge_index[1].astype(jnp.int32)

    pad = E_PAD - N_EDGES
    fill = jnp.full((pad,), N_NODES, jnp.int32)
    src_flat = jnp.concatenate([src, fill])
    dst_flat = jnp.concatenate([dst, fill])
    srcw = src_flat.reshape(NW, NCHUNK, K)
    dstw = dst_flat.reshape(NW, NCHUNK, K)

    feat_pad = jnp.zeros((NPAD, D), jnp.float32).at[:N_NODES].set(features)
    zrow = jnp.zeros((NPAD, D), jnp.float32)
    z1 = jnp.zeros((NPAD,), jnp.float32)

    agg, degp = _aggregate(feat_pad, src_flat, dst_flat, zrow, z1)
    degp = degp.reshape(NC, NPAD)

    wp2 = W_pred.reshape(2, D)
    bc2 = b_conv.reshape(1, D)
    s_arr, t_arr = _dense(feat_pad, agg, degp, W_self, W_neigh, bc2, wp2,
                          b_pred)

    logits = _edge_logits(s_arr, t_arr, srcw, dstw)
    return logits.reshape(-1)[:N_EDGES]
